# Initial kernel scaffold; baseline (speedup 1.0000x reference)
import jax
import jax.numpy as jnp
from jax.experimental import pallas as pl


def kernel(z, embed_weight):
    raise NotImplementedError("write your pallas kernel here")


# speedup vs baseline: 1.0121x; 1.0121x over previous
import jax
import jax.numpy as jnp
from jax import lax
from jax.experimental import pallas as pl

K = 8192
D = 32
BETA_C = 0.25
TN = 512
KC = 4096
EPS = 1e-10


def _vq_body(z_ref, e_ref, zq_ref, idx_ref, loss_ref, perp_ref, counts_ref):
    i = pl.program_id(0)
    nsteps = pl.num_programs(0)
    n_total = TN * nsteps

    @pl.when(i == 0)
    def _init():
        counts_ref[...] = jnp.zeros_like(counts_ref)
        loss_ref[...] = jnp.zeros((1, 1), jnp.float32)
        perp_ref[...] = jnp.zeros((1, 1), jnp.float32)

    zb = z_ref[...]
    zn = jnp.sum(zb * zb, axis=1, keepdims=True)

    bestq = None
    bidx = None
    for c in range(K // KC):
        e_c = e_ref[pl.ds(c * KC, KC), :]
        en = jnp.sum(e_c * e_c, axis=1)[None, :]
        s = lax.dot_general(zb, e_c, (((1,), (1,)), ((), ())),
                            preferred_element_type=jnp.float32)
        d = (zn + en) - 2.0 * s
        m = jnp.min(d, axis=1, keepdims=True)
        iota = lax.broadcasted_iota(jnp.int32, (TN, KC), 1) + c * KC
        li = jnp.min(jnp.where(d <= m, iota, jnp.int32(2 ** 30)),
                     axis=1, keepdims=True)
        if c == 0:
            best = m
            bidx = li
        else:
            take = m < bestq
            best = jnp.where(take, m, bestq)
            bidx = jnp.where(take, li, bidx)
        bestq = best.astype(jnp.bfloat16).astype(jnp.float32)

    zq = jnp.zeros((TN, D), jnp.float32)
    for c in range(K // KC):
        e_c = e_ref[pl.ds(c * KC, KC), :]
        iota = lax.broadcasted_iota(jnp.int32, (TN, KC), 1) + c * KC
        oh = (iota == bidx).astype(jnp.float32)
        zq = zq + lax.dot_general(oh, e_c, (((1,), (0,)), ((), ())),
                                  preferred_element_type=jnp.float32)
        counts_ref[0:1, pl.ds(c * KC, KC)] += jnp.sum(oh, axis=0, keepdims=True)

    zqst = zb + (zq - zb)
    zq_ref[...] = zqst
    idx_ref[...] = bidx
    loss_ref[...] += jnp.sum((zqst - zb) ** 2, keepdims=True)

    @pl.when(i == nsteps - 1)
    def _finalize():
        loss_ref[...] = loss_ref[...] * (BETA_C / (n_total * D))
        p = counts_ref[...] * (1.0 / n_total)
        ent = jnp.sum(p * jnp.log(p + EPS), keepdims=True)
        perp_ref[...] = jnp.exp(-ent)


def kernel(z, embed_weight):
    zf = z.reshape(-1, D)
    n = zf.shape[0]
    nb = n // TN
    zq, idx, loss, perp, _counts = pl.pallas_call(
        _vq_body,
        grid=(nb,),
        in_specs=[
            pl.BlockSpec((TN, D), lambda i: (i, 0)),
            pl.BlockSpec((K, D), lambda i: (0, 0)),
        ],
        out_specs=[
            pl.BlockSpec((TN, D), lambda i: (i, 0)),
            pl.BlockSpec((TN, 1), lambda i: (i, 0)),
            pl.BlockSpec((1, 1), lambda i: (0, 0)),
            pl.BlockSpec((1, 1), lambda i: (0, 0)),
            pl.BlockSpec((1, K), lambda i: (0, 0)),
        ],
        out_shape=[
            jax.ShapeDtypeStruct((n, D), jnp.float32),
            jax.ShapeDtypeStruct((n, 1), jnp.int32),
            jax.ShapeDtypeStruct((1, 1), jnp.float32),
            jax.ShapeDtypeStruct((1, 1), jnp.float32),
            jax.ShapeDtypeStruct((1, K), jnp.float32),
        ],
    )(zf, embed_weight)
    return (zq.reshape(z.shape), loss[0, 0], idx[:, 0], perp[0, 0])
